# Initial kernel scaffold; baseline (speedup 1.0000x reference)
import jax
import jax.numpy as jnp
from jax.experimental import pallas as pl


def kernel(x, edge_index, weight, W1, b1, W2, b2, W3, b3):
    raise NotImplementedError("write your pallas kernel here")


# speedup vs baseline: 8.3867x; 8.3867x over previous
import functools

import jax
import jax.numpy as jnp
from jax import lax
from jax.experimental import pallas as pl
from jax.experimental.pallas import tpu as pltpu
from jax.experimental.pallas import tpu_sc as plsc

_N = 10000
_E = 320000
_D = 128
_C = 40
_DP = 64
_ALPHA = 0.9
_NC = 2
_NS = 16
_L = 16
_NW = _NC * _NS
_EPW = _E // _NW
_CH = 80
_NCH = _EPW // _CH
_EPS = _E // _NS
_NCH2 = _EPS // _CH
_NP = 10240
_RPTP = _NP // _NS

_mesh = plsc.VectorSubcoreMesh(core_axis_name="c", subcore_axis_name="s")
_sc_params = pltpu.CompilerParams(needs_layout_passes=False, use_tc_tiling_on_sc=False)


@functools.partial(
    pl.kernel,
    out_type=jax.ShapeDtypeStruct((_NC, _NP), jnp.float32),
    mesh=_mesh,
    compiler_params=_sc_params,
    scratch_types=[
        pltpu.VMEM((_NCH, _CH), jnp.int32),
        pltpu.VMEM((_NCH, _CH), jnp.float32),
        pltpu.VMEM_SHARED((_NP,), jnp.float32),
    ],
)
def _deg_kernel(col_hbm, w_hbm, z1_hbm, degp_hbm, col_v, w_v, acc_sh):
    cid = lax.axis_index("c")
    sid = lax.axis_index("s")
    wid = cid * _NS + sid
    pltpu.sync_copy(z1_hbm, acc_sh.at[pl.ds(sid * _RPTP, _RPTP)])
    pltpu.sync_copy(col_hbm.at[wid], col_v)
    pltpu.sync_copy(w_hbm.at[wid], w_v)
    plsc.subcore_barrier()

    def body(j, carry):
        pltpu.sync_copy(w_v.at[j], acc_sh.at[col_v.at[j]], add=True)
        return carry

    lax.fori_loop(0, _NCH, body, 0)
    plsc.subcore_barrier()
    pltpu.sync_copy(acc_sh.at[pl.ds(sid * _RPTP, _RPTP)],
                    degp_hbm.at[cid, pl.ds(sid * _RPTP, _RPTP)])


@functools.partial(
    pl.kernel,
    out_type=jax.ShapeDtypeStruct((_NW, _NCH, _CH), jnp.float32),
    mesh=_mesh,
    compiler_params=_sc_params,
    scratch_types=[
        pltpu.VMEM((_NP,), jnp.float32),
        pltpu.VMEM((_NCH, _CH), jnp.int32),
        pltpu.VMEM((_NCH, _CH), jnp.int32),
        pltpu.VMEM((_NCH, _CH), jnp.float32),
        pltpu.VMEM((_NCH, _CH), jnp.float32),
    ],
)
def _norm_kernel(dinv_hbm, row_hbm, col_hbm, w_hbm, norm_hbm,
                 dinv_v, row_v, col_v, w_v, norm_v):
    cid = lax.axis_index("c")
    sid = lax.axis_index("s")
    wid = cid * _NS + sid
    pltpu.sync_copy(dinv_hbm, dinv_v)
    pltpu.sync_copy(row_hbm.at[wid], row_v)
    pltpu.sync_copy(col_hbm.at[wid], col_v)
    pltpu.sync_copy(w_hbm.at[wid], w_v)

    def ebody(j, carry):
        for q in range(_CH // _L):
            sl = pl.ds(q * _L, _L)
            r = row_v[j, sl]
            c = col_v[j, sl]
            dr = plsc.load_gather(dinv_v, [r])
            dc = plsc.load_gather(dinv_v, [c])
            norm_v[j, sl] = dr * w_v[j, sl] * dc
        return carry

    lax.fori_loop(0, _NCH, ebody, 0)
    pltpu.sync_copy(norm_v, norm_hbm.at[wid])


def _make_agg(dvh):
    @functools.partial(
        pl.kernel,
        out_type=jax.ShapeDtypeStruct((_NC, _NP, dvh), jnp.float32),
        mesh=_mesh,
        compiler_params=_sc_params,
        scratch_types=[
            pltpu.VMEM((_NCH2, _CH), jnp.int32),
            pltpu.VMEM((_NCH2, _CH), jnp.int32),
            pltpu.VMEM((_NCH2, _CH), jnp.float32),
            pltpu.VMEM((_CH, dvh), jnp.float32),
            pltpu.VMEM_SHARED((_NP, dvh), jnp.float32),
            pltpu.SemaphoreType.DMA,
        ],
    )
    def _agg(y0_hbm, y1_hbm, row_hbm, col_hbm, norm_hbm, z_hbm, out_hbm,
             row_v, col_v, norm_v, buf, acc_sh, sem):
        cid = lax.axis_index("c")
        sid = lax.axis_index("s")
        pltpu.sync_copy(z_hbm, acc_sh.at[pl.ds(sid * _RPTP, _RPTP)])
        pltpu.sync_copy(row_hbm.at[sid], row_v)
        pltpu.sync_copy(col_hbm.at[sid], col_v)
        pltpu.sync_copy(norm_hbm.at[sid], norm_v)
        plsc.subcore_barrier()

        def run(y_hbm):
            def chunk(j, carry):
                pltpu.async_copy(y_hbm.at[row_v.at[j]], buf, sem).wait()

                def edge(b, c2):
                    jb = jnp.full((_L,), j, jnp.int32)
                    bb = jnp.full((_L,), b, jnp.int32)
                    s = plsc.load_gather(norm_v, [jb, bb])
                    for k in range(dvh // _L):
                        sl = pl.ds(k * _L, _L)
                        buf[b, sl] = buf[b, sl] * s
                    return c2

                lax.fori_loop(0, _CH, edge, 0)
                pltpu.sync_copy(buf, acc_sh.at[col_v.at[j]], add=True)
                return carry

            lax.fori_loop(0, _NCH2, chunk, 0)

        @pl.when(cid == 0)
        def _():
            run(y0_hbm)

        @pl.when(cid == 1)
        def _():
            run(y1_hbm)

        plsc.subcore_barrier()
        pltpu.sync_copy(acc_sh.at[pl.ds(sid * _RPTP, _RPTP)],
                        out_hbm.at[cid, pl.ds(sid * _RPTP, _RPTP)])

    return _agg


_agg64 = _make_agg(_D // 2)
_agg32 = _make_agg(_DP // 2)


_BLK = 2000


def _elu(t):
    return jnp.where(t > 0, t, jnp.exp(jnp.minimum(t, 0.0)) - 1.0)


def _combine(y_full, p_ref, dv_ref):
    dv = dv_ref[...]
    scale = _ALPHA + (1.0 - _ALPHA) * dv * dv
    agg = jnp.concatenate([p_ref[0], p_ref[1]], axis=-1)
    return scale * y_full + (1.0 - _ALPHA) * agg


def _tc_a_kern(x_ref, p_ref, dv_ref, w1_ref, b1_ref, w2_ref, o0_ref, o1_ref):
    z = _combine(x_ref[...], p_ref, dv_ref)
    t = jnp.dot(z, w1_ref[...], preferred_element_type=jnp.float32,
                precision=lax.Precision.HIGHEST) + b1_ref[...]
    h = _elu(t)
    y = jnp.dot(h, w2_ref[...], preferred_element_type=jnp.float32,
                precision=lax.Precision.HIGHEST)
    o0_ref[...] = y[:, :_D // 2]
    o1_ref[...] = y[:, _D // 2:]


def _tc_b_kern(y0_ref, y1_ref, p_ref, dv_ref, b2_ref, w3_ref, o0_ref, o1_ref):
    y_full = jnp.concatenate([y0_ref[...], y1_ref[...]], axis=-1)
    z = _combine(y_full, p_ref, dv_ref)
    h = _elu(z + b2_ref[...])
    y = jnp.dot(h, w3_ref[...], preferred_element_type=jnp.float32,
                precision=lax.Precision.HIGHEST)
    o0_ref[...] = y[:, :_DP // 2]
    o1_ref[...] = y[:, _DP // 2:]


def _tc_c_kern(y0_ref, y1_ref, p_ref, dv_ref, b3_ref, o_ref):
    y_full = jnp.concatenate([y0_ref[...], y1_ref[...]], axis=-1)
    o_ref[...] = _combine(y_full, p_ref, dv_ref) + b3_ref[...]


def _tc_dinv_kern(degp_ref, o_ref):
    o_ref[...] = lax.rsqrt(1.0 + degp_ref[0:1, :] + degp_ref[1:2, :])


def _tc_dinv(degp):
    return pl.pallas_call(
        _tc_dinv_kern,
        out_shape=jax.ShapeDtypeStruct((1, _NP), jnp.float32),
    )(degp)


def _row_spec(w):
    return pl.BlockSpec((_BLK, w), lambda i: (i, 0))


def _p_spec(w):
    return pl.BlockSpec((2, _BLK, w), lambda i: (0, i, 0))


def _full_spec(a, b):
    return pl.BlockSpec((a, b), lambda i: (0, 0))


def _tc_a(x, p, dinv_col, W1, b1, W2):
    return pl.pallas_call(
        _tc_a_kern,
        grid=(_N // _BLK,),
        in_specs=[_row_spec(_D), _p_spec(_D // 2), _row_spec(1),
                  _full_spec(_D, _D), _full_spec(1, _D), _full_spec(_D, _D)],
        out_specs=[_row_spec(_D // 2), _row_spec(_D // 2)],
        out_shape=[jax.ShapeDtypeStruct((_N, _D // 2), jnp.float32),
                   jax.ShapeDtypeStruct((_N, _D // 2), jnp.float32)],
    )(x, p, dinv_col, W1, b1, W2)


def _tc_b(y0, y1, p, dinv_col, b2, W3p):
    return pl.pallas_call(
        _tc_b_kern,
        grid=(_N // _BLK,),
        in_specs=[_row_spec(_D // 2), _row_spec(_D // 2), _p_spec(_D // 2),
                  _row_spec(1), _full_spec(1, _D), _full_spec(_D, _DP)],
        out_specs=[_row_spec(_DP // 2), _row_spec(_DP // 2)],
        out_shape=[jax.ShapeDtypeStruct((_N, _DP // 2), jnp.float32),
                   jax.ShapeDtypeStruct((_N, _DP // 2), jnp.float32)],
    )(y0, y1, p, dinv_col, b2, W3p)


def _tc_c(y0, y1, p, dinv_col, b3p):
    return pl.pallas_call(
        _tc_c_kern,
        grid=(_N // _BLK,),
        in_specs=[_row_spec(_DP // 2), _row_spec(_DP // 2), _p_spec(_DP // 2),
                  _row_spec(1), _full_spec(1, _DP)],
        out_specs=_row_spec(_DP),
        out_shape=jax.ShapeDtypeStruct((_N, _DP), jnp.float32),
    )(y0, y1, p, dinv_col, b3p)


def kernel(x, edge_index, weight, W1, b1, W2, b2, W3, b3):
    row = edge_index[0]
    col = edge_index[1]
    roww = row.reshape(_NW, _NCH, _CH)
    colw = col.reshape(_NW, _NCH, _CH)
    ww = weight.reshape(_NW, _NCH, _CH)
    rows = row.reshape(_NS, _NCH2, _CH)
    cols = col.reshape(_NS, _NCH2, _CH)
    z1 = jnp.zeros((_RPTP,), jnp.float32)
    z64 = jnp.zeros((_RPTP, _D // 2), jnp.float32)
    z32 = jnp.zeros((_RPTP, _DP // 2), jnp.float32)
    W3p = jnp.zeros((_D, _DP), jnp.float32).at[:, :_C].set(W3)
    b3p = jnp.zeros((1, _DP), jnp.float32).at[:, :_C].set(b3)

    degp = _deg_kernel(colw, ww, z1)
    dinv = _tc_dinv(degp).reshape(_NP)
    normw = _norm_kernel(dinv, roww, colw, ww)
    norms = normw.reshape(_NS, _NCH2, _CH)
    dinv_col = dinv[:_N].reshape(_N, 1)

    x0 = x[:, :_D // 2]
    x1 = x[:, _D // 2:]
    p1 = _agg64(x0, x1, rows, cols, norms, z64)
    y20, y21 = _tc_a(x, p1, dinv_col, W1, b1.reshape(1, _D), W2)
    p2 = _agg64(y20, y21, rows, cols, norms, z64)
    y30, y31 = _tc_b(y20, y21, p2, dinv_col, b2.reshape(1, _D), W3p)
    p3 = _agg32(y30, y31, rows, cols, norms, z32)
    out64 = _tc_c(y30, y31, p3, dinv_col, b3p)
    return out64[:, :_C]
